# baseline (speedup 1.0000x reference)
import functools

import jax
import jax.numpy as jnp
from jax import lax
from jax.experimental import pallas as pl
from jax.experimental.pallas import tpu as pltpu
from jax.experimental.pallas import tpu_sc as plsc

N = 16384
EMB = 33
D_HALF = 32

NUM_CORES = 2
NUM_SUBCORES = 16
NW = NUM_CORES * NUM_SUBCORES
B_PER_W = N // NW
IDX_CHUNK = 128
N_CHUNKS = B_PER_W // IDX_CHUNK
L = 16


_sc_mesh = plsc.VectorSubcoreMesh(core_axis_name="c", subcore_axis_name="s")


@functools.partial(
    pl.kernel,
    mesh=_sc_mesh,
    out_type=jax.ShapeDtypeStruct((N, D_HALF), jnp.float32),
    compiler_params=pltpu.CompilerParams(
        use_tc_tiling_on_sc=False, needs_layout_passes=False
    ),
    scratch_types=[
        pltpu.VMEM((B_PER_W, 1), jnp.float32),
        pltpu.VMEM((B_PER_W,), jnp.int32),
        pltpu.VMEM((B_PER_W, D_HALF), jnp.float32),
        pltpu.SemaphoreType.DMA,
    ],
)
def _gather_sc(table_hbm, token_hbm, out_hbm, col_v, idx_v, rows_v, sem):
    wid = lax.axis_index("s") * NUM_CORES + lax.axis_index("c")
    base = wid * B_PER_W
    pltpu.sync_copy(token_hbm.at[pl.ds(base, B_PER_W), pl.ds(0, 1)], col_v)
    zeros = lax.iota(jnp.int32, L) * 0
    for g in range(B_PER_W // L):
        row_idx = lax.iota(jnp.int32, L) + (g * L)
        vals = plsc.load_gather(col_v, [row_idx, zeros])
        idx_v[pl.ds(g * L, L)] = vals.astype(jnp.int32)
    copies = []
    for j in range(N_CHUNKS):
        copies.append(
            pltpu.async_copy(
                table_hbm.at[idx_v.at[pl.ds(j * IDX_CHUNK, IDX_CHUNK)]],
                rows_v.at[pl.ds(j * IDX_CHUNK, IDX_CHUNK)],
                sem,
            )
        )
    for c in copies:
        c.wait()
    pltpu.sync_copy(rows_v, out_hbm.at[pl.ds(base, B_PER_W)])


def _fuse_body(tok_ref, emb_ref, w_ref, b_ref, out_ref):
    x = tok_ref[:, 1:EMB]
    y = jnp.dot(x, w_ref[:].T, preferred_element_type=jnp.float32)
    proj = jnp.maximum(y + b_ref[:], 0.0)
    out_ref[:] = jnp.concatenate([emb_ref[:], proj], axis=1)


_BLK = 2048


def _fuse_tc(token, emb, W, b):
    b2 = b.reshape(1, D_HALF)
    return pl.pallas_call(
        _fuse_body,
        grid=(N // _BLK,),
        in_specs=[
            pl.BlockSpec((_BLK, EMB), lambda i: (i, 0)),
            pl.BlockSpec((_BLK, D_HALF), lambda i: (i, 0)),
            pl.BlockSpec((D_HALF, EMB - 1), lambda i: (0, 0)),
            pl.BlockSpec((1, D_HALF), lambda i: (0, 0)),
        ],
        out_specs=pl.BlockSpec((_BLK, 2 * D_HALF), lambda i: (i, 0)),
        out_shape=jax.ShapeDtypeStruct((N, 2 * D_HALF), jnp.float32),
    )(token, emb, W, b2)


def kernel(token, table, W, b):
    emb = _gather_sc(table, token)
    return _fuse_tc(token, emb, W, b)


# speedup vs baseline: 1.3702x; 1.3702x over previous
import functools

import jax
import jax.numpy as jnp
from jax import lax
from jax.experimental import pallas as pl
from jax.experimental.pallas import tpu as pltpu
from jax.experimental.pallas import tpu_sc as plsc

N = 16384
EMB = 33
D_HALF = 32

NUM_CORES = 2
NUM_SUBCORES = 16
NW = NUM_CORES * NUM_SUBCORES
B_PER_W = N // NW
L = 16
GROUPS = B_PER_W // L


_sc_mesh = plsc.VectorSubcoreMesh(core_axis_name="c", subcore_axis_name="s")


@functools.partial(
    pl.kernel,
    mesh=_sc_mesh,
    out_type=jax.ShapeDtypeStruct((N, D_HALF), jnp.float32),
    scratch_types=[
        pltpu.VMEM((B_PER_W,), jnp.int32),
        pltpu.VMEM((B_PER_W, D_HALF), jnp.float32),
        pltpu.SemaphoreType.DMA,
    ],
)
def _gather_sc(table_hbm, idx_hbm, out_hbm, idx_v, rows_v, sem):
    wid = lax.axis_index("s") * NUM_CORES + lax.axis_index("c")
    base = wid * B_PER_W
    pltpu.sync_copy(idx_hbm.at[pl.ds(base, B_PER_W)], idx_v)

    def issue_group(g):
        vec = idx_v[pl.ds(g * L, L)]
        for i in range(L):
            pltpu.async_copy(
                table_hbm.at[pl.ds(vec[i], 1), :],
                rows_v.at[pl.ds(g * L + i, 1), :],
                sem,
            )

    def wait_group(_):
        for _i in range(L):
            pltpu.make_async_copy(
                table_hbm.at[pl.ds(0, 1), :],
                rows_v.at[pl.ds(0, 1), :],
                sem,
            ).wait()

    issue_group(0)

    def body(g, _):
        issue_group(g)
        wait_group(g - 1)
        return 0

    lax.fori_loop(1, GROUPS, body, 0)
    wait_group(GROUPS - 1)
    pltpu.sync_copy(rows_v, out_hbm.at[pl.ds(base, B_PER_W)])


def _fuse_body(tok_ref, emb_ref, w_ref, b_ref, out_ref):
    x = tok_ref[:, 1:EMB]
    y = jnp.dot(x, w_ref[:].T, preferred_element_type=jnp.float32)
    proj = jnp.maximum(y + b_ref[:], 0.0)
    out_ref[:] = jnp.concatenate([emb_ref[:], proj], axis=1)


_BLK = 2048


def _fuse_tc(token, emb, W, b):
    b2 = b.reshape(1, D_HALF)
    return pl.pallas_call(
        _fuse_body,
        grid=(N // _BLK,),
        in_specs=[
            pl.BlockSpec((_BLK, EMB), lambda i: (i, 0)),
            pl.BlockSpec((_BLK, D_HALF), lambda i: (i, 0)),
            pl.BlockSpec((D_HALF, EMB - 1), lambda i: (0, 0)),
            pl.BlockSpec((1, D_HALF), lambda i: (0, 0)),
        ],
        out_specs=pl.BlockSpec((_BLK, 2 * D_HALF), lambda i: (i, 0)),
        out_shape=jax.ShapeDtypeStruct((N, 2 * D_HALF), jnp.float32),
    )(token, emb, W, b2)


def kernel(token, table, W, b):
    idx = token[:, 0].astype(jnp.int32)
    emb = _gather_sc(table, idx)
    return _fuse_tc(token, emb, W, b)


# speedup vs baseline: 2.4052x; 1.7553x over previous
import functools

import jax
import jax.numpy as jnp
from jax import lax
from jax.experimental import pallas as pl
from jax.experimental.pallas import tpu as pltpu
from jax.experimental.pallas import tpu_sc as plsc

N = 16384
EMB = 33
D_HALF = 32
V = 100000

NUM_CORES = 2
NUM_SUBCORES = 16
NW = NUM_CORES * NUM_SUBCORES
L = 16
CHUNK = 2048
N_CHUNKS = N // CHUNK


_sc_mesh = plsc.VectorSubcoreMesh(core_axis_name="c", subcore_axis_name="s")


@functools.partial(
    pl.kernel,
    mesh=_sc_mesh,
    out_type=jax.ShapeDtypeStruct((D_HALF, N), jnp.float32),
    compiler_params=pltpu.CompilerParams(needs_layout_passes=False),
    scratch_types=[
        pltpu.VMEM((1, V), jnp.float32),
        pltpu.VMEM((CHUNK,), jnp.int32),
        pltpu.VMEM((1, CHUNK), jnp.float32),
    ],
)
def _gather_sc(tableT_hbm, idx_hbm, embT_hbm, row_v, idx_v, out_v):
    wid = lax.axis_index("s") * NUM_CORES + lax.axis_index("c")
    pltpu.sync_copy(tableT_hbm.at[pl.ds(wid, 1), :], row_v)
    zeros = lax.iota(jnp.int32, L) * 0

    def chunk_body(c, _):
        pltpu.sync_copy(idx_hbm.at[pl.ds(c * CHUNK, CHUNK)], idx_v)

        def group_body(g, _):
            base = g * L
            iv = idx_v[pl.ds(base, L)]
            out_v[0, pl.ds(base, L)] = plsc.load_gather(row_v, [zeros, iv])
            return 0

        lax.fori_loop(0, CHUNK // L, group_body, 0)
        pltpu.sync_copy(
            out_v, embT_hbm.at[pl.ds(wid, 1), pl.ds(c * CHUNK, CHUNK)]
        )
        return 0

    lax.fori_loop(0, N_CHUNKS, chunk_body, 0)


def _fuse_body(tokT_ref, embT_ref, w_ref, b_ref, out_ref):
    x = tokT_ref[1:EMB, :]
    y = jnp.dot(w_ref[:], x, preferred_element_type=jnp.float32)
    proj = jnp.maximum(y + b_ref[:], 0.0)
    out_ref[:] = jnp.concatenate([embT_ref[:], proj], axis=0)


_BLK = 2048


def _fuse_tc(tokenT, embT, W, b):
    b2 = b.reshape(D_HALF, 1)
    return pl.pallas_call(
        _fuse_body,
        grid=(N // _BLK,),
        in_specs=[
            pl.BlockSpec((EMB, _BLK), lambda i: (0, i)),
            pl.BlockSpec((D_HALF, _BLK), lambda i: (0, i)),
            pl.BlockSpec((D_HALF, EMB - 1), lambda i: (0, 0)),
            pl.BlockSpec((D_HALF, 1), lambda i: (0, 0)),
        ],
        out_specs=pl.BlockSpec((2 * D_HALF, _BLK), lambda i: (0, i)),
        out_shape=jax.ShapeDtypeStruct((2 * D_HALF, N), jnp.float32),
    )(tokenT, embT, W, b2)


def kernel(token, table, W, b):
    tokenT = token.T
    tableT = table.T
    idx = tokenT[0, :].astype(jnp.int32)
    embT = _gather_sc(tableT, idx)
    outT = _fuse_tc(tokenT, embT, W, b)
    return outT.T


# speedup vs baseline: 2.7764x; 1.1543x over previous
import functools

import jax
import jax.numpy as jnp
from jax import lax
from jax.experimental import pallas as pl
from jax.experimental.pallas import tpu as pltpu
from jax.experimental.pallas import tpu_sc as plsc

N = 16384
EMB = 33
D_HALF = 32
V = 100000

NUM_CORES = 2
NUM_SUBCORES = 16
NW = NUM_CORES * NUM_SUBCORES
L = 16
CHUNK = 2048
N_CHUNKS = N // CHUNK


_sc_mesh = plsc.VectorSubcoreMesh(core_axis_name="c", subcore_axis_name="s")


@functools.partial(
    pl.kernel,
    mesh=_sc_mesh,
    out_type=jax.ShapeDtypeStruct((D_HALF, N), jnp.float32),
    compiler_params=pltpu.CompilerParams(needs_layout_passes=False),
    scratch_types=[
        pltpu.VMEM((1, V), jnp.float32),
        pltpu.VMEM((N,), jnp.int32),
        pltpu.VMEM((1, CHUNK), jnp.float32),
        pltpu.SemaphoreType.DMA,
    ],
)
def _gather_sc(tableT_hbm, idx_hbm, embT_hbm, row_v, idx_v, out_v, sem):
    wid = lax.axis_index("s") * NUM_CORES + lax.axis_index("c")
    row_cp = pltpu.async_copy(tableT_hbm.at[pl.ds(wid, 1), :], row_v, sem)
    idx_cp = pltpu.async_copy(idx_hbm, idx_v, sem)
    row_cp.wait()
    idx_cp.wait()
    zeros = lax.iota(jnp.int32, L) * 0
    UNROLL = 8

    for c in range(N_CHUNKS):

        def group_body(gg, _, c=c):
            base = gg * (L * UNROLL)
            for u in range(UNROLL):
                off = base + u * L
                iv = idx_v[pl.ds(c * CHUNK + off, L)]
                out_v[0, pl.ds(off, L)] = plsc.load_gather(
                    row_v, [zeros, iv]
                )
            return 0

        lax.fori_loop(0, CHUNK // (L * UNROLL), group_body, 0)
        pltpu.sync_copy(
            out_v, embT_hbm.at[pl.ds(wid, 1), pl.ds(c * CHUNK, CHUNK)]
        )


def _fuse_body(tokT_ref, embT_ref, w_ref, b_ref, out_ref):
    x = tokT_ref[1:EMB, :]
    y = jnp.dot(w_ref[:], x, preferred_element_type=jnp.float32)
    proj = jnp.maximum(y + b_ref[:], 0.0)
    out_ref[:] = jnp.concatenate([embT_ref[:], proj], axis=0)


_BLK = 2048


def _fuse_tc(tokenT, embT, W, b):
    b2 = b.reshape(D_HALF, 1)
    return pl.pallas_call(
        _fuse_body,
        grid=(N // _BLK,),
        in_specs=[
            pl.BlockSpec((EMB, _BLK), lambda i: (0, i)),
            pl.BlockSpec((D_HALF, _BLK), lambda i: (0, i)),
            pl.BlockSpec((D_HALF, EMB - 1), lambda i: (0, 0)),
            pl.BlockSpec((D_HALF, 1), lambda i: (0, 0)),
        ],
        out_specs=pl.BlockSpec((2 * D_HALF, _BLK), lambda i: (0, i)),
        out_shape=jax.ShapeDtypeStruct((2 * D_HALF, N), jnp.float32),
    )(tokenT, embT, W, b2)


def kernel(token, table, W, b):
    tokenT = token.T
    tableT = table.T
    idx = tokenT[0, :].astype(jnp.int32)
    embT = _gather_sc(tableT, idx)
    outT = _fuse_tc(tokenT, embT, W, b)
    return outT.T
